# Initial kernel scaffold; baseline (speedup 1.0000x reference)
import jax
import jax.numpy as jnp
from jax.experimental import pallas as pl


def kernel(x, edge_index, Wg0, bg0, Wl0, bl0, g0, be0, Wg1, bg1, Wl1, bl1, g1, be1, Wout, bout):
    raise NotImplementedError("write your pallas kernel here")


# speedup vs baseline: 13.3107x; 13.3107x over previous
import functools

import jax
import jax.numpy as jnp
from jax import lax
from jax.experimental import pallas as pl
from jax.experimental.pallas import tpu as pltpu
from jax.experimental.pallas import tpu_sc as plsc

N = 10000
E = 320000
D = 128
DH = D // 2
EPS = 1e-5

NC = 2
NS = 16
NW = NC * NS

RPS = N // NS
ZR = 125

NPD = 10240
RPSD = NPD // NS

EPW = E // NW
C = 80
NCHUNK = EPW // C

BM = 1000
GRID = N // BM

_mesh = plsc.VectorSubcoreMesh(core_axis_name="c", subcore_axis_name="s")
_f32 = jnp.float32


@functools.partial(
    pl.kernel,
    out_type=jax.ShapeDtypeStruct((NC, NPD), _f32),
    mesh=_mesh,
    scratch_types=[
        pltpu.VMEM((NCHUNK, C), jnp.int32),
        pltpu.VMEM((C,), _f32),
        pltpu.VMEM((RPSD,), _f32),
        pltpu.VMEM_SHARED((NPD,), _f32),
    ],
)
def _deg_kernel(dst_hbm, out_hbm, dstv, onesv, zv, acc):
    c = lax.axis_index("c")
    s = lax.axis_index("s")
    wid = c * NS + s

    for j in range(C // 16):
        onesv[pl.ds(j * 16, 16)] = jnp.ones((16,), _f32)

    def zfill(i, carry):
        zv[pl.ds(i * 16, 16)] = jnp.zeros((16,), _f32)
        return carry

    lax.fori_loop(0, RPSD // 16, zfill, 0)
    pltpu.sync_copy(zv, acc.at[pl.ds(s * RPSD, RPSD)])
    plsc.subcore_barrier()

    pltpu.sync_copy(dst_hbm.at[wid], dstv)

    def chunk(j, carry):
        pltpu.sync_copy(onesv, acc.at[dstv.at[j]], add=True)
        return carry

    lax.fori_loop(0, NCHUNK, chunk, 0)
    plsc.subcore_barrier()

    pltpu.sync_copy(acc.at[pl.ds(s * RPSD, RPSD)],
                    out_hbm.at[c, pl.ds(s * RPSD, RPSD)])


@functools.partial(
    pl.kernel,
    out_type=jax.ShapeDtypeStruct((NC, 2, N, DH), _f32),
    mesh=_mesh,
    scratch_types=[
        pltpu.VMEM((NCHUNK, C), jnp.int32),
        pltpu.VMEM((NCHUNK, C), jnp.int32),
        pltpu.VMEM((C, DH), _f32),
        pltpu.VMEM((ZR, DH), _f32),
        pltpu.VMEM_SHARED((N, DH), _f32),
        pltpu.SemaphoreType.DMA,
    ],
    compiler_params=pltpu.CompilerParams(use_tc_tiling_on_sc=False),
)
def _scatter_kernel(ylo_hbm, yhi_hbm, src_hbm, dst_hbm, out_hbm, srcv, dstv,
                    rows, zbuf, acc, sem):
    c = lax.axis_index("c")
    s = lax.axis_index("s")
    wid = c * NS + s

    def zfill(i, carry):
        for j in range(DH // 16):
            zbuf[i, pl.ds(j * 16, 16)] = jnp.zeros((16,), _f32)
        return carry

    lax.fori_loop(0, ZR, zfill, 0)

    pltpu.sync_copy(src_hbm.at[wid], srcv)
    pltpu.sync_copy(dst_hbm.at[wid], dstv)

    for h, y_hbm in ((0, ylo_hbm), (1, yhi_hbm)):
        def zcopy(i, carry):
            pltpu.sync_copy(zbuf, acc.at[pl.ds(s * RPS + i * ZR, ZR), :])
            return carry

        lax.fori_loop(0, RPS // ZR, zcopy, 0)
        plsc.subcore_barrier()

        def chunk(j, carry):
            pltpu.async_copy(y_hbm.at[srcv.at[j]], rows, sem).wait()
            pltpu.sync_copy(rows, acc.at[dstv.at[j]], add=True)
            return carry

        lax.fori_loop(0, NCHUNK, chunk, 0)
        plsc.subcore_barrier()

        def ocopy(i, carry):
            r0 = s * RPS + i * ZR
            pltpu.sync_copy(acc.at[pl.ds(r0, ZR), :],
                            out_hbm.at[c, h, pl.ds(r0, ZR), :])
            return carry

        lax.fori_loop(0, RPS // ZR, ocopy, 0)


_HI = lax.Precision.HIGHEST


def _rows(i):
    return (i, 0)


def _full(*z):
    def im(i):
        return z
    return im


def _tc_pre(deg_t, x, Wg, Wl, bl):
    def body(deg_ref, x_ref, wg_ref, wl_ref, bl_ref, dinv_ref, ylo_ref,
             yhi_ref, x2_ref):
        deg = deg_ref[:, 0:1] + deg_ref[:, 1:2] + 1.0
        dinv = lax.rsqrt(deg)
        dinv_ref[...] = dinv
        xv = x_ref[...]
        y = jnp.dot(xv, wg_ref[...], precision=_HI) * dinv
        ylo_ref[...] = y[:, :DH]
        yhi_ref[...] = y[:, DH:]
        x2_ref[...] = jnp.dot(xv, wl_ref[...], precision=_HI) + bl_ref[...]

    return pl.pallas_call(
        body,
        grid=(GRID,),
        in_specs=[
            pl.BlockSpec((BM, 2), _rows),
            pl.BlockSpec((BM, D), _rows),
            pl.BlockSpec((D, D), _full(0, 0)),
            pl.BlockSpec((D, D), _full(0, 0)),
            pl.BlockSpec((1, D), _full(0, 0)),
        ],
        out_specs=(
            pl.BlockSpec((BM, 1), _rows),
            pl.BlockSpec((BM, DH), _rows),
            pl.BlockSpec((BM, DH), _rows),
            pl.BlockSpec((BM, D), _rows),
        ),
        out_shape=(
            jax.ShapeDtypeStruct((N, 1), _f32),
            jax.ShapeDtypeStruct((N, DH), _f32),
            jax.ShapeDtypeStruct((N, DH), _f32),
            jax.ShapeDtypeStruct((N, D), _f32),
        ),
    )(deg_t, x, Wg, Wl, bl)


def _tc_sum(sp, ylo, yhi, x2, dinv, bg):
    def body(sp_ref, ylo_ref, yhi_ref, x2_ref, dinv_ref, bg_ref,
             h_ref, sums_ref):
        i = pl.program_id(0)
        S = jnp.concatenate(
            [sp_ref[0, 0] + sp_ref[1, 0], sp_ref[0, 1] + sp_ref[1, 1]], axis=1)
        y = jnp.concatenate([ylo_ref[...], yhi_ref[...]], axis=1)
        h = dinv_ref[...] * (S + y) + bg_ref[...] + x2_ref[...]
        h_ref[...] = h
        blk = jnp.concatenate(
            [jnp.sum(h, axis=0, keepdims=True),
             jnp.sum(h * h, axis=0, keepdims=True)], axis=0)

        @pl.when(i == 0)
        def _():
            sums_ref[...] = blk

        @pl.when(i > 0)
        def _():
            sums_ref[...] += blk

    return pl.pallas_call(
        body,
        grid=(GRID,),
        in_specs=[
            pl.BlockSpec((NC, 2, BM, DH), lambda i: (0, 0, i, 0)),
            pl.BlockSpec((BM, DH), _rows),
            pl.BlockSpec((BM, DH), _rows),
            pl.BlockSpec((BM, D), _rows),
            pl.BlockSpec((BM, 1), _rows),
            pl.BlockSpec((1, D), _full(0, 0)),
        ],
        out_specs=(
            pl.BlockSpec((BM, D), _rows),
            pl.BlockSpec((2, D), _full(0, 0)),
        ),
        out_shape=(
            jax.ShapeDtypeStruct((N, D), _f32),
            jax.ShapeDtypeStruct((2, D), _f32),
        ),
    )(sp, ylo, yhi, x2, dinv, bg)


def _bn_elu(h_ref, sums_ref, g_ref, be_ref):
    mu = sums_ref[0:1, :] * (1.0 / N)
    var = sums_ref[1:2, :] * (1.0 / N) - mu * mu
    hn = (h_ref[...] - mu) * lax.rsqrt(var + EPS) * g_ref[...] + be_ref[...]
    return jnp.where(hn > 0, hn, jnp.exp(hn) - 1.0)


def _tc_mid(h, sums, g, be, dinv, Wg, Wl, bl):
    def body(h_ref, sums_ref, g_ref, be_ref, dinv_ref, wg_ref, wl_ref, bl_ref,
             y1lo_ref, y1hi_ref, x21_ref):
        h1 = _bn_elu(h_ref, sums_ref, g_ref, be_ref)
        y1 = jnp.dot(h1, wg_ref[...], precision=_HI) * dinv_ref[...]
        y1lo_ref[...] = y1[:, :DH]
        y1hi_ref[...] = y1[:, DH:]
        x21_ref[...] = jnp.dot(h1, wl_ref[...], precision=_HI) + bl_ref[...]

    return pl.pallas_call(
        body,
        grid=(GRID,),
        in_specs=[
            pl.BlockSpec((BM, D), _rows),
            pl.BlockSpec((2, D), _full(0, 0)),
            pl.BlockSpec((1, D), _full(0, 0)),
            pl.BlockSpec((1, D), _full(0, 0)),
            pl.BlockSpec((BM, 1), _rows),
            pl.BlockSpec((D, D), _full(0, 0)),
            pl.BlockSpec((D, D), _full(0, 0)),
            pl.BlockSpec((1, D), _full(0, 0)),
        ],
        out_specs=(
            pl.BlockSpec((BM, DH), _rows),
            pl.BlockSpec((BM, DH), _rows),
            pl.BlockSpec((BM, D), _rows),
        ),
        out_shape=(
            jax.ShapeDtypeStruct((N, DH), _f32),
            jax.ShapeDtypeStruct((N, DH), _f32),
            jax.ShapeDtypeStruct((N, D), _f32),
        ),
    )(h, sums, g, be, dinv, Wg, Wl, bl)


def _tc_out(h, sums, g, be, WoutT, bout):
    def body(h_ref, sums_ref, g_ref, be_ref, wo_ref, bo_ref, out_ref):
        h2 = _bn_elu(h_ref, sums_ref, g_ref, be_ref)
        out_ref[...] = (jnp.sum(h2 * wo_ref[...], axis=1, keepdims=True)
                        + bo_ref[...])

    return pl.pallas_call(
        body,
        grid=(GRID,),
        in_specs=[
            pl.BlockSpec((BM, D), _rows),
            pl.BlockSpec((2, D), _full(0, 0)),
            pl.BlockSpec((1, D), _full(0, 0)),
            pl.BlockSpec((1, D), _full(0, 0)),
            pl.BlockSpec((1, D), _full(0, 0)),
            pl.BlockSpec((1, 1), _full(0, 0)),
        ],
        out_specs=pl.BlockSpec((BM, 1), _rows),
        out_shape=jax.ShapeDtypeStruct((N, 1), _f32),
    )(h, sums, g, be, WoutT, bout)


def kernel(x, edge_index, Wg0, bg0, Wl0, bl0, g0, be0,
           Wg1, bg1, Wl1, bl1, g1, be1, Wout, bout):
    src = edge_index[0].reshape(NW, NCHUNK, C)
    dst = edge_index[1].reshape(NW, NCHUNK, C)

    degp = _deg_kernel(dst)
    deg_t = jnp.transpose(degp)[:N, :]

    dinv, y0lo, y0hi, x20 = _tc_pre(deg_t, x, Wg0, Wl0, bl0.reshape(1, D))
    sp0 = _scatter_kernel(y0lo, y0hi, src, dst)
    h1, sums1 = _tc_sum(sp0, y0lo, y0hi, x20, dinv, bg0.reshape(1, D))
    y1lo, y1hi, x21 = _tc_mid(h1, sums1, g0.reshape(1, D), be0.reshape(1, D),
                              dinv, Wg1, Wl1, bl1.reshape(1, D))
    sp1 = _scatter_kernel(y1lo, y1hi, src, dst)
    h2, sums2 = _tc_sum(sp1, y1lo, y1hi, x21, dinv, bg1.reshape(1, D))
    return _tc_out(h2, sums2, g1.reshape(1, D), be1.reshape(1, D),
                   Wout.reshape(1, D), bout.reshape(1, 1))
